# Initial kernel scaffold; baseline (speedup 1.0000x reference)
import jax
import jax.numpy as jnp
from jax.experimental import pallas as pl


def kernel(time_idx, time_embedding):
    raise NotImplementedError("write your pallas kernel here")


# speedup vs baseline: 1.1017x; 1.1017x over previous
import functools

import jax
import jax.numpy as jnp
from jax import lax
from jax.experimental import pallas as pl
from jax.experimental.pallas import tpu as pltpu
from jax.experimental.pallas import tpu_sc as plsc

_B = 16384
_S = 50
_D = 32
_B_TOTAL = _B * _S
_NW = 32
_B_PER_W = _B_TOTAL // _NW
_CHUNK = 1600
_N_CHUNKS = _B_PER_W // _CHUNK


def _sc_gather(idx_flat, table):
    mesh = plsc.VectorSubcoreMesh(core_axis_name="c", subcore_axis_name="s")

    @functools.partial(
        pl.kernel,
        mesh=mesh,
        out_type=jax.ShapeDtypeStruct((_B_TOTAL, _D), jnp.float32),
        scratch_types=[
            pltpu.VMEM((_CHUNK,), jnp.int32),
            pltpu.VMEM((_CHUNK, _D), jnp.float32),
            pltpu.SemaphoreType.DMA,
        ],
        compiler_params=pltpu.CompilerParams(use_tc_tiling_on_sc=False),
    )
    def k(idx_hbm, table_hbm, out_hbm, idx_v, rows_v, sem):
        wid = lax.axis_index("s") * 2 + lax.axis_index("c")
        base = wid * _B_PER_W

        def body(i, carry):
            off = base + i * _CHUNK
            pltpu.sync_copy(idx_hbm.at[pl.ds(off, _CHUNK)], idx_v)
            pltpu.async_copy(table_hbm.at[idx_v], rows_v, sem).wait()
            pltpu.sync_copy(rows_v, out_hbm.at[pl.ds(off, _CHUNK)])
            return carry

        lax.fori_loop(0, _N_CHUNKS, body, jnp.int32(0))

    return k(idx_flat, table)


def kernel(time_idx, time_embedding):
    idx_flat = time_idx.reshape(-1).astype(jnp.int32)
    out = _sc_gather(idx_flat, time_embedding)
    return out.reshape(_B, _S, _D)


# speedup vs baseline: 1.1125x; 1.0098x over previous
import functools

import jax
import jax.numpy as jnp
from jax import lax
from jax.experimental import pallas as pl
from jax.experimental.pallas import tpu as pltpu
from jax.experimental.pallas import tpu_sc as plsc

_B = 16384
_S = 50
_D = 32
_B_TOTAL = _B * _S
_NW = 32
_B_PER_W = _B_TOTAL // _NW
_CHUNK = 1600
_N_CHUNKS = _B_PER_W // _CHUNK
_NBUF = 2


def _sc_gather(idx_flat, table):
    mesh = plsc.VectorSubcoreMesh(core_axis_name="c", subcore_axis_name="s")

    @functools.partial(
        pl.kernel,
        mesh=mesh,
        out_type=jax.ShapeDtypeStruct((_B_TOTAL, _D), jnp.float32),
        scratch_types=[
            [pltpu.VMEM((_CHUNK,), jnp.int32) for _ in range(_NBUF)],
            [pltpu.VMEM((_CHUNK, _D), jnp.float32) for _ in range(_NBUF)],
            [pltpu.SemaphoreType.DMA for _ in range(_NBUF)],
            [pltpu.SemaphoreType.DMA for _ in range(_NBUF)],
            [pltpu.SemaphoreType.DMA for _ in range(_NBUF)],
        ],
        compiler_params=pltpu.CompilerParams(use_tc_tiling_on_sc=False),
    )
    def k(idx_hbm, table_hbm, out_hbm, idx_v, rows_v, isem, gsem, ssem):
        wid = lax.axis_index("s") * 2 + lax.axis_index("c")
        base = wid * _B_PER_W

        def chunk_slice(i):
            return pl.ds(base + i * _CHUNK, _CHUNK)

        for b in range(_NBUF):
            pltpu.async_copy(idx_hbm.at[chunk_slice(b)], idx_v[b], isem[b])
        for b in range(_NBUF):
            pltpu.make_async_copy(idx_hbm.at[chunk_slice(b)], idx_v[b],
                                  isem[b]).wait()
            pltpu.async_copy(table_hbm.at[idx_v[b]], rows_v[b], gsem[b])

        for i in range(_N_CHUNKS):
            b = i % _NBUF
            pltpu.make_async_copy(table_hbm.at[idx_v[b]], rows_v[b],
                                  gsem[b]).wait()
            pltpu.async_copy(rows_v[b], out_hbm.at[chunk_slice(i)], ssem[b])
            nxt = i + _NBUF
            if nxt < _N_CHUNKS:
                pltpu.async_copy(idx_hbm.at[chunk_slice(nxt)], idx_v[b],
                                 isem[b])
                pltpu.make_async_copy(rows_v[b], out_hbm.at[chunk_slice(i)],
                                      ssem[b]).wait()
                pltpu.make_async_copy(idx_hbm.at[chunk_slice(nxt)], idx_v[b],
                                      isem[b]).wait()
                pltpu.async_copy(table_hbm.at[idx_v[b]], rows_v[b], gsem[b])
            else:
                pltpu.make_async_copy(rows_v[b], out_hbm.at[chunk_slice(i)],
                                      ssem[b]).wait()

    return k(idx_flat, table)


def kernel(time_idx, time_embedding):
    idx_flat = time_idx.reshape(-1).astype(jnp.int32)
    out = _sc_gather(idx_flat, time_embedding)
    return out.reshape(_B, _S, _D)


# speedup vs baseline: 1.8015x; 1.6193x over previous
import functools

import jax
import jax.numpy as jnp
from jax import lax
from jax.experimental import pallas as pl
from jax.experimental.pallas import tpu as pltpu
from jax.experimental.pallas import tpu_sc as plsc

_B = 16384
_S = 50
_D = 32
_NW = 32
_ROWS_W = _B // _NW
_CR = 32
_N_CHUNKS = _ROWS_W // _CR
_NBUF = 2


def _sc_gather(idx, table):
    mesh = plsc.VectorSubcoreMesh(core_axis_name="c", subcore_axis_name="s")

    @functools.partial(
        pl.kernel,
        mesh=mesh,
        out_type=jax.ShapeDtypeStruct((_B, _S, _D), jnp.float32),
        scratch_types=[
            [pltpu.VMEM((_CR * _S,), jnp.int32) for _ in range(_NBUF)],
            [pltpu.VMEM((_CR * _S, _D), jnp.float32) for _ in range(_NBUF)],
            [pltpu.SemaphoreType.DMA for _ in range(_NBUF)],
            [pltpu.SemaphoreType.DMA for _ in range(_NBUF)],
            [pltpu.SemaphoreType.DMA for _ in range(_NBUF)],
        ],
        compiler_params=pltpu.CompilerParams(use_tc_tiling_on_sc=False),
    )
    def k(idx_hbm, table_hbm, out_hbm, idx_v, rows_v, isem, gsem, ssem):
        wid = lax.axis_index("s") * 2 + lax.axis_index("c")
        row_base = wid * _ROWS_W

        def rows(i):
            return pl.ds(row_base + i * _CR, _CR)

        def idxs(i):
            return pl.ds((row_base + i * _CR) * _S, _CR * _S)

        for b in range(_NBUF):
            pltpu.async_copy(idx_hbm.at[idxs(b)], idx_v[b], isem[b])
        for b in range(_NBUF):
            pltpu.make_async_copy(idx_hbm.at[idxs(b)], idx_v[b],
                                  isem[b]).wait()
            pltpu.async_copy(table_hbm.at[idx_v[b]], rows_v[b], gsem[b])

        for i in range(_N_CHUNKS):
            b = i % _NBUF
            pltpu.make_async_copy(table_hbm.at[idx_v[b]], rows_v[b],
                                  gsem[b]).wait()
            for j in range(_CR):
                pltpu.async_copy(rows_v[b].at[pl.ds(j * _S, _S), :],
                                 out_hbm.at[row_base + i * _CR + j], ssem[b])
            nxt = i + _NBUF
            if nxt < _N_CHUNKS:
                pltpu.async_copy(idx_hbm.at[idxs(nxt)], idx_v[b], isem[b])
                for j in range(_CR):
                    pltpu.make_async_copy(
                        rows_v[b].at[pl.ds(j * _S, _S), :],
                        out_hbm.at[row_base + i * _CR + j], ssem[b]).wait()
                pltpu.make_async_copy(idx_hbm.at[idxs(nxt)], idx_v[b],
                                      isem[b]).wait()
                pltpu.async_copy(table_hbm.at[idx_v[b]], rows_v[b], gsem[b])
            else:
                for j in range(_CR):
                    pltpu.make_async_copy(
                        rows_v[b].at[pl.ds(j * _S, _S), :],
                        out_hbm.at[row_base + i * _CR + j], ssem[b]).wait()

    return k(idx, table)


def kernel(time_idx, time_embedding):
    idx_flat = time_idx.reshape(-1).astype(jnp.int32)
    return _sc_gather(idx_flat, time_embedding)
